# baseline (speedup 1.0000x reference)
import functools

import jax
import jax.numpy as jnp
from jax.experimental import pallas as pl
from jax.experimental.pallas import tpu as pltpu

S, B, D, E, K, F = 2048, 1, 1024, 8, 2, 2048
N = S * B
EP = E + 1
LANES = 128


TT = 512
NT = N // TT


def _moe_kernel(x_ref, wg_ref, bg_ref, w1_ref, b1_ref, w2_ref, b2_ref,
                out_ref, combine_s, xbf_s):
    e = pl.program_id(1)

    @pl.when(e == 0)
    def _gate():
        xb = x_ref[...]
        logits = jax.lax.dot_general(
            xb.astype(jnp.bfloat16), wg_ref[...].astype(jnp.bfloat16),
            (((1,), (0,)), ((), ())),
            preferred_element_type=jnp.float32) + bg_ref[0]
        lane = jax.lax.broadcasted_iota(jnp.int32, (TT, LANES), 1)
        neg = jnp.float32(-1e30)
        logm = jnp.where(lane < E, logits, neg)
        m1 = jnp.max(logm, axis=1, keepdims=True)
        i1 = jnp.min(jnp.where(logm == m1, lane, LANES), axis=1, keepdims=True)
        logm2 = jnp.where(lane == i1, neg, logm)
        m2 = jnp.max(logm2, axis=1, keepdims=True)
        i2 = jnp.min(jnp.where(logm2 == m2, lane, LANES), axis=1, keepdims=True)
        d = jnp.exp(m2 - m1)
        wa = 1.0 / (1.0 + d)
        wb = d / (1.0 + d)
        comb = jnp.where(lane == i1, wa, 0.0) + jnp.where(lane == i2, wb, 0.0)
        comb = comb + jnp.where(lane == E, jnp.float32(1.0 / K), 0.0)
        combine_s[...] = comb
        xbf_s[...] = xb.astype(jnp.bfloat16)

    xb16 = xbf_s[...]
    h = jax.lax.dot_general(
        xb16, w1_ref[0], (((1,), (0,)), ((), ())),
        preferred_element_type=jnp.float32) + b1_ref[0, 0]
    h = jnp.maximum(h, 0.0).astype(jnp.bfloat16)
    ye = jax.lax.dot_general(
        h, w2_ref[0], (((1,), (0,)), ((), ())),
        preferred_element_type=jnp.float32) + b2_ref[0, 0]

    lane = jax.lax.broadcasted_iota(jnp.int32, (TT, LANES), 1)
    c = jnp.sum(jnp.where(lane == e, combine_s[...], 0.0), axis=1,
                keepdims=True)

    @pl.when(e == 0)
    def _init():
        out_ref[...] = c * ye

    @pl.when(e != 0)
    def _acc():
        out_ref[...] = out_ref[...] + c * ye


@functools.partial(jax.jit, static_argnames=())
def kernel(x, Wg, bg, W1, b1, W2, b2, Ws1, bs1, Ws2, bs2):
    xf = x.reshape(N, D)
    wgp = jnp.zeros((D, LANES), jnp.float32).at[:, :E].set(Wg)
    bgp = jnp.zeros((1, LANES), jnp.float32).at[0, :E].set(bg)
    w1p = jnp.concatenate([W1, Ws1[None]], axis=0).astype(jnp.bfloat16)
    b1p = jnp.concatenate([b1, bs1[None]], axis=0).reshape(EP, 1, F)
    w2p = jnp.concatenate([W2, Ws2[None]], axis=0).astype(jnp.bfloat16)
    b2p = jnp.concatenate([b2, bs2[None]], axis=0).reshape(EP, 1, D)

    out = pl.pallas_call(
        _moe_kernel,
        grid=(NT, EP),
        in_specs=[
            pl.BlockSpec((TT, D), lambda t, e: (t, 0)),
            pl.BlockSpec((D, LANES), lambda t, e: (0, 0)),
            pl.BlockSpec((1, LANES), lambda t, e: (0, 0)),
            pl.BlockSpec((1, D, F), lambda t, e: (e, 0, 0)),
            pl.BlockSpec((1, 1, F), lambda t, e: (e, 0, 0)),
            pl.BlockSpec((1, F, D), lambda t, e: (e, 0, 0)),
            pl.BlockSpec((1, 1, D), lambda t, e: (e, 0, 0)),
        ],
        out_specs=pl.BlockSpec((TT, D), lambda t, e: (t, 0)),
        out_shape=jax.ShapeDtypeStruct((N, D), jnp.float32),
        scratch_shapes=[
            pltpu.VMEM((TT, LANES), jnp.float32),
            pltpu.VMEM((TT, D), jnp.bfloat16),
        ],
        compiler_params=pltpu.CompilerParams(
            dimension_semantics=("arbitrary", "arbitrary"),
        ),
    )(xf, wgp, bgp, w1p, b1p, w2p, b2p)
    return out.reshape(S, B, D)


# speedup vs baseline: 1.0150x; 1.0150x over previous
import functools

import jax
import jax.numpy as jnp
from jax.experimental import pallas as pl
from jax.experimental.pallas import tpu as pltpu

S, B, D, E, K, F = 2048, 1, 1024, 8, 2, 2048
N = S * B
EP = E + 1
LANES = 128


TT = 1024
NT = N // TT


def _moe_kernel(x_ref, wg_ref, bg_ref, w1_ref, b1_ref, w2_ref, b2_ref,
                out_ref, combine_s, xbf_s):
    e = pl.program_id(1)

    @pl.when(e == 0)
    def _gate():
        xb = x_ref[...]
        logits = jax.lax.dot_general(
            xb.astype(jnp.bfloat16), wg_ref[...].astype(jnp.bfloat16),
            (((1,), (0,)), ((), ())),
            preferred_element_type=jnp.float32) + bg_ref[0]
        lane = jax.lax.broadcasted_iota(jnp.int32, (TT, LANES), 1)
        neg = jnp.float32(-1e30)
        logm = jnp.where(lane < E, logits, neg)
        m1 = jnp.max(logm, axis=1, keepdims=True)
        i1 = jnp.min(jnp.where(logm == m1, lane, LANES), axis=1, keepdims=True)
        logm2 = jnp.where(lane == i1, neg, logm)
        m2 = jnp.max(logm2, axis=1, keepdims=True)
        i2 = jnp.min(jnp.where(logm2 == m2, lane, LANES), axis=1, keepdims=True)
        d = jnp.exp(m2 - m1)
        wa = 1.0 / (1.0 + d)
        wb = d / (1.0 + d)
        comb = jnp.where(lane == i1, wa, 0.0) + jnp.where(lane == i2, wb, 0.0)
        comb = comb + jnp.where(lane == E, jnp.float32(1.0 / K), 0.0)
        combine_s[...] = comb
        xbf_s[...] = xb.astype(jnp.bfloat16)

    xb16 = xbf_s[...]
    h = jax.lax.dot_general(
        xb16, w1_ref[0], (((1,), (0,)), ((), ())),
        preferred_element_type=jnp.float32) + b1_ref[0, 0]
    h = jnp.maximum(h, 0.0).astype(jnp.bfloat16)
    ye = jax.lax.dot_general(
        h, w2_ref[0], (((1,), (0,)), ((), ())),
        preferred_element_type=jnp.float32) + b2_ref[0, 0]

    lane = jax.lax.broadcasted_iota(jnp.int32, (TT, LANES), 1)
    c = jnp.sum(jnp.where(lane == e, combine_s[...], 0.0), axis=1,
                keepdims=True)

    @pl.when(e == 0)
    def _init():
        out_ref[...] = c * ye

    @pl.when(e != 0)
    def _acc():
        out_ref[...] = out_ref[...] + c * ye


@functools.partial(jax.jit, static_argnames=())
def kernel(x, Wg, bg, W1, b1, W2, b2, Ws1, bs1, Ws2, bs2):
    xf = x.reshape(N, D)
    wgp = jnp.zeros((D, LANES), jnp.float32).at[:, :E].set(Wg)
    bgp = jnp.zeros((1, LANES), jnp.float32).at[0, :E].set(bg)
    w1p = jnp.concatenate([W1, Ws1[None]], axis=0).astype(jnp.bfloat16)
    b1p = jnp.concatenate([b1, bs1[None]], axis=0).reshape(EP, 1, F)
    w2p = jnp.concatenate([W2, Ws2[None]], axis=0).astype(jnp.bfloat16)
    b2p = jnp.concatenate([b2, bs2[None]], axis=0).reshape(EP, 1, D)

    out = pl.pallas_call(
        _moe_kernel,
        grid=(NT, EP),
        in_specs=[
            pl.BlockSpec((TT, D), lambda t, e: (t, 0)),
            pl.BlockSpec((D, LANES), lambda t, e: (0, 0)),
            pl.BlockSpec((1, LANES), lambda t, e: (0, 0)),
            pl.BlockSpec((1, D, F), lambda t, e: (e, 0, 0)),
            pl.BlockSpec((1, 1, F), lambda t, e: (e, 0, 0)),
            pl.BlockSpec((1, F, D), lambda t, e: (e, 0, 0)),
            pl.BlockSpec((1, 1, D), lambda t, e: (e, 0, 0)),
        ],
        out_specs=pl.BlockSpec((TT, D), lambda t, e: (t, 0)),
        out_shape=jax.ShapeDtypeStruct((N, D), jnp.float32),
        scratch_shapes=[
            pltpu.VMEM((TT, LANES), jnp.float32),
            pltpu.VMEM((TT, D), jnp.bfloat16),
        ],
        compiler_params=pltpu.CompilerParams(
            dimension_semantics=("arbitrary", "arbitrary"),
        ),
    )(xf, wgp, bgp, w1p, b1p, w2p, b2p)
    return out.reshape(S, B, D)
